# baseline (speedup 1.0000x reference)
import functools

import jax
import jax.numpy as jnp
from jax import lax
from jax.experimental import pallas as pl
from jax.experimental.pallas import tpu as pltpu
from jax.experimental.pallas import tpu_sc as plsc

_COL_GROUPS = 8
_ROW_GROUPS = 4


def _bag_body(L, w_hbm, part_hbm, w_v, out_v):
    core = lax.axis_index("c")
    sub = lax.axis_index("s")
    wid = sub * 2 + core
    colg = wid % _COL_GROUPS
    rowg = wid // _COL_GROUPS
    rows = L // _ROW_GROUPS
    c0 = colg * 128
    r0 = rowg * rows

    pltpu.sync_copy(w_hbm.at[pl.ds(r0, rows), pl.ds(c0, 128)], w_v)

    scale = jnp.float32(1.0 / L)
    zero = jnp.zeros((16,), jnp.float32)

    def body(i, accs):
        return tuple(accs[v] + w_v[i, pl.ds(v * 16, 16)] for v in range(8))

    accs = lax.fori_loop(0, rows, body, (zero,) * 8)
    for v in range(8):
        out_v[pl.ds(v * 16, 16)] = accs[v] * scale

    pltpu.sync_copy(out_v, part_hbm.at[pl.ds(rowg * L + c0, 128)])


def _sc_partials(W, L):
    mesh = plsc.VectorSubcoreMesh(core_axis_name="c", subcore_axis_name="s")
    return pl.kernel(
        functools.partial(_bag_body, L),
        out_type=jax.ShapeDtypeStruct((_ROW_GROUPS * L,), jnp.float32),
        mesh=mesh,
        scratch_types=[
            pltpu.VMEM((L // _ROW_GROUPS, 128), jnp.float32),
            pltpu.VMEM((128,), jnp.float32),
        ],
    )(W)


def _add_body(x_ref, part_ref, o_ref):
    bag = jnp.sum(part_ref[...], axis=0, keepdims=True)
    o_ref[...] = x_ref[...] + bag


def _tc_add(x2d, partials, block_rows):
    rows, dim = x2d.shape
    grid = (rows // block_rows,)
    return pl.pallas_call(
        _add_body,
        grid=grid,
        in_specs=[
            pl.BlockSpec((block_rows, dim), lambda i: (i, 0)),
            pl.BlockSpec((_ROW_GROUPS, dim), lambda i: (0, 0)),
        ],
        out_specs=pl.BlockSpec((block_rows, dim), lambda i: (i, 0)),
        out_shape=jax.ShapeDtypeStruct((rows, dim), jnp.float32),
        compiler_params=pltpu.CompilerParams(
            dimension_semantics=("parallel",),
        ),
    )(x2d, partials)


def kernel(x, W):
    B, S, L = x.shape
    partials = _sc_partials(W, L).reshape(_ROW_GROUPS, L)
    x2d = x.reshape(B * S, L)
    half = (B * S) // 2
    out0 = _tc_add(x2d[:half], partials, block_rows=2048)
    out1 = _tc_add(x2d[half:], partials, block_rows=2048)
    out = jnp.concatenate([out0, out1], axis=0)
    return out.reshape(B, S, L)


# speedup vs baseline: 1.6048x; 1.6048x over previous
import functools

import jax
import jax.numpy as jnp
from jax import lax
from jax.experimental import pallas as pl
from jax.experimental.pallas import tpu as pltpu
from jax.experimental.pallas import tpu_sc as plsc

_CHUNK = 16


def _body(L, R, x_hbm, w_hbm, out_hbm,
          wbuf, stage, pA, pB, inb0, inb1, outb0, outb1, spart,
          sin0, sin1, sout0, sout1):
    core = lax.axis_index("c")
    sid = lax.axis_index("s")
    wid = sid * 2 + core

    colg = wid % 2
    rowg = wid // 2
    cb = pl.multiple_of(colg * (L // 2), 128)
    rows_per_tile = R // 16
    r0x = rowg * rows_per_tile
    nchunks = rows_per_tile // _CHUNK

    def in_slice(g):
        row = pl.multiple_of(r0x + g * _CHUNK, 8)
        return x_hbm.at[pl.ds(row, _CHUNK), pl.ds(cb, L // 2)]

    def out_slice(g):
        row = pl.multiple_of(r0x + g * _CHUNK, 8)
        return out_hbm.at[pl.ds(row, _CHUNK), pl.ds(cb, L // 2)]

    pltpu.async_copy(in_slice(0), inb0, sin0)
    pltpu.async_copy(in_slice(1), inb1, sin1)

    colg8 = sid % 8
    rowg2 = sid // 8
    wc0 = pl.multiple_of(colg8 * 128, 128)
    wr0 = pl.multiple_of(rowg2 * (L // 2), 8)
    pltpu.sync_copy(w_hbm.at[pl.ds(wr0, L // 2), pl.ds(wc0, 128)], wbuf)

    def acc_body(i, accs):
        return tuple(accs[v] + wbuf[i, pl.ds(v * 16, 16)] for v in range(8))

    accs = lax.fori_loop(0, L // 2, acc_body, (jnp.zeros((16,), jnp.float32),) * 8)
    scale = jnp.float32(1.0 / L)
    for v in range(8):
        stage[pl.ds(v * 16, 16)] = accs[v] * scale
    soff = pl.multiple_of(rowg2 * L + wc0, 8)
    pltpu.sync_copy(stage, spart.at[pl.ds(soff, 128)])
    plsc.subcore_barrier()

    sA = pl.multiple_of(cb, 8)
    sB = pl.multiple_of(L + cb, 8)
    pltpu.sync_copy(spart.at[pl.ds(sA, L // 2)], pA)
    pltpu.sync_copy(spart.at[pl.ds(sB, L // 2)], pB)
    bagv = tuple(pA[pl.ds(j * 16, 16)] + pB[pl.ds(j * 16, 16)] for j in range(32))

    nvec = (L // 2) // 16

    def outer(s, carry):
        for b, (ib, ob, si, so) in enumerate(
                ((inb0, outb0, sin0, sout0), (inb1, outb1, sin1, sout1))):
            g = 2 * s + b

            @pl.when(g >= 2)
            def _drain():
                pltpu.make_async_copy(ob, out_slice(g), so).wait()

            pltpu.make_async_copy(in_slice(g), ib, si).wait()
            for r in range(_CHUNK):
                for j in range(nvec):
                    ob[r, pl.ds(j * 16, 16)] = ib[r, pl.ds(j * 16, 16)] + bagv[j]
            pltpu.async_copy(ob, out_slice(g), so)

            @pl.when(g + 2 < nchunks)
            def _prefetch():
                pltpu.async_copy(in_slice(g + 2), ib, si)
        return carry

    lax.fori_loop(0, nchunks // 2, outer, 0)
    pltpu.make_async_copy(outb0, out_slice(nchunks - 2), sout0).wait()
    pltpu.make_async_copy(outb1, out_slice(nchunks - 1), sout1).wait()


def _sc_full(x2d, W, L):
    R = x2d.shape[0]
    mesh = plsc.VectorSubcoreMesh(core_axis_name="c", subcore_axis_name="s")
    return pl.kernel(
        functools.partial(_body, L, R),
        out_type=jax.ShapeDtypeStruct((R, L), jnp.float32),
        mesh=mesh,
        scratch_types=[
            pltpu.VMEM((L // 2, 128), jnp.float32),
            pltpu.VMEM((128,), jnp.float32),
            pltpu.VMEM((L // 2,), jnp.float32),
            pltpu.VMEM((L // 2,), jnp.float32),
            pltpu.VMEM((_CHUNK, L // 2), jnp.float32),
            pltpu.VMEM((_CHUNK, L // 2), jnp.float32),
            pltpu.VMEM((_CHUNK, L // 2), jnp.float32),
            pltpu.VMEM((_CHUNK, L // 2), jnp.float32),
            pltpu.VMEM_SHARED((2 * L,), jnp.float32),
            pltpu.SemaphoreType.DMA,
            pltpu.SemaphoreType.DMA,
            pltpu.SemaphoreType.DMA,
            pltpu.SemaphoreType.DMA,
        ],
    )(x2d, W)


def kernel(x, W):
    B, S, L = x.shape
    x2d = x.reshape(B * S, L)
    out = _sc_full(x2d, W, L)
    return out.reshape(B, S, L)
